# Initial kernel scaffold; baseline (speedup 1.0000x reference)
import jax
import jax.numpy as jnp
from jax.experimental import pallas as pl


def kernel(features, sparse_adj, embeddings, W_conv, W_mlp):
    raise NotImplementedError("write your pallas kernel here")


# speedup vs baseline: 10.4530x; 10.4530x over previous
import functools

import jax
import jax.numpy as jnp
from jax import lax
from jax.experimental import pallas as pl
from jax.experimental.pallas import tpu as pltpu
from jax.experimental.pallas import tpu_sc as plsc

N = 10000
E = 320000
D = 128
KAPPA = 0.95

NC = 2
NS = 16
NW = NC * NS
CHUNK = 128
EPAD = 327680
CPT = EPAD // (NW * CHUNK)
ACC_ROWS = 10240
DUMMY = N
DEG_W = 16
ROWS_PER_TILE = ACC_ROWS // NS

_mesh = plsc.VectorSubcoreMesh(core_axis_name="c", subcore_axis_name="s")


@functools.partial(
    pl.kernel,
    out_type=jax.ShapeDtypeStruct((NC, ACC_ROWS, DEG_W), jnp.float32),
    mesh=_mesh,
    scratch_types=[
        pltpu.VMEM((CHUNK, DEG_W), jnp.float32),
        pltpu.VMEM((CHUNK, DEG_W), jnp.float32),
        pltpu.VMEM((CHUNK,), jnp.int32),
        pltpu.VMEM_SHARED((ACC_ROWS, DEG_W), jnp.float32),
    ],
)
def _sc_degree(dst_hbm, aux_hbm, out_hbm, zero_v, ones_v, idx_v, acc):
    c = lax.axis_index("c")
    s = lax.axis_index("s")
    wid = s * NC + c
    pltpu.sync_copy(aux_hbm.at[0], zero_v)
    pltpu.sync_copy(aux_hbm.at[1], ones_v)
    base = s * ROWS_PER_TILE
    for k in range(ROWS_PER_TILE // CHUNK):
        pltpu.sync_copy(zero_v, acc.at[pl.ds(base + k * CHUNK, CHUNK)])
    plsc.subcore_barrier()

    def body(j, carry):
        pltpu.sync_copy(dst_hbm.at[wid, j], idx_v)
        pltpu.sync_copy(ones_v, acc.at[idx_v], add=True)
        return carry

    lax.fori_loop(0, CPT, body, 0)
    plsc.subcore_barrier()
    pltpu.sync_copy(acc.at[pl.ds(base, ROWS_PER_TILE)],
                    out_hbm.at[c, pl.ds(base, ROWS_PER_TILE)])


@functools.partial(
    pl.kernel,
    out_type=jax.ShapeDtypeStruct((NC, ACC_ROWS, D), jnp.float32),
    mesh=_mesh,
    scratch_types=[
        pltpu.VMEM((CHUNK, D), jnp.float32),
        pltpu.VMEM((CHUNK,), jnp.int32),
        pltpu.VMEM((CHUNK,), jnp.int32),
        pltpu.VMEM((CHUNK, D), jnp.float32),
        pltpu.VMEM_SHARED((ACC_ROWS, D), jnp.float32),
        pltpu.SemaphoreType.DMA,
    ],
)
def _sc_aggregate(src_hbm, dst_hbm, sxw_hbm, zeros_hbm, out_hbm,
                  zero_v, isrc_v, idst_v, rows_v, acc, sem):
    c = lax.axis_index("c")
    s = lax.axis_index("s")
    wid = s * NC + c
    pltpu.sync_copy(zeros_hbm, zero_v)
    base = s * ROWS_PER_TILE
    for k in range(ROWS_PER_TILE // CHUNK):
        pltpu.sync_copy(zero_v, acc.at[pl.ds(base + k * CHUNK, CHUNK)])
    plsc.subcore_barrier()

    def body(j, carry):
        pltpu.sync_copy(src_hbm.at[wid, j], isrc_v)
        pltpu.sync_copy(dst_hbm.at[wid, j], idst_v)
        pltpu.async_copy(sxw_hbm.at[isrc_v], rows_v, sem).wait()
        pltpu.sync_copy(rows_v, acc.at[idst_v], add=True)
        return carry

    lax.fori_loop(0, CPT, body, 0)
    plsc.subcore_barrier()
    pltpu.sync_copy(acc.at[pl.ds(base, ROWS_PER_TILE)],
                    out_hbm.at[c, pl.ds(base, ROWS_PER_TILE)])


def _project(W):
    Wabs = jnp.abs(W)
    row_sum = jnp.sum(Wabs, axis=1, keepdims=True)
    hi0 = jnp.max(Wabs, axis=1, keepdims=True)

    def bis(i, lohi):
        lo, hi = lohi
        mid = 0.5 * (lo + hi)
        g = jnp.sum(jnp.maximum(Wabs - mid, 0.0), axis=1, keepdims=True)
        gt = g > KAPPA
        return (jnp.where(gt, mid, lo), jnp.where(gt, hi, mid))

    lo, hi = lax.fori_loop(0, 40, bis, (jnp.zeros_like(hi0), hi0))
    theta = 0.5 * (lo + hi)
    proj = jnp.sign(W) * jnp.maximum(Wabs - theta, 0.0)
    return jnp.where(row_sum > KAPPA, proj, W)


def _dinv_from_parts(degp_ref):
    deg = degp_ref[0, 0:N, 0:1] + degp_ref[1, 0:N, 0:1] + 1.0
    return lax.rsqrt(deg)


def _tc_prepare_body(emb_ref, w_ref, degp_ref, out_ref):
    Wc = _project(w_ref[...])
    dinv = _dinv_from_parts(degp_ref)
    xw = lax.dot_general(emb_ref[...], Wc, (((1,), (1,)), ((), ())),
                         preferred_element_type=jnp.float32)
    out_ref[...] = xw * dinv


def _tc_finish_body(parts_ref, sxw_ref, feat_ref, wmlp_ref, degp_ref, out_ref):
    dinv = _dinv_from_parts(degp_ref)
    agg = parts_ref[0, 0:N] + parts_ref[1, 0:N] + sxw_ref[...]
    mlp = lax.dot_general(feat_ref[...], wmlp_ref[...],
                          (((1,), (1,)), ((), ())),
                          preferred_element_type=jnp.float32)
    out_ref[...] = jnp.maximum(agg * dinv + mlp, 0.0)


def kernel(features, sparse_adj, embeddings, W_conv, W_mlp):
    src = sparse_adj[0]
    dst = sparse_adj[1]
    pad = EPAD - E
    src_p = jnp.concatenate([src, jnp.zeros((pad,), jnp.int32)])
    dst_p = jnp.concatenate([dst, jnp.full((pad,), DUMMY, jnp.int32)])
    src3 = src_p.reshape(NW, CPT, CHUNK)
    dst3 = dst_p.reshape(NW, CPT, CHUNK)

    aux16 = jnp.stack([jnp.zeros((CHUNK, DEG_W), jnp.float32),
                       jnp.ones((CHUNK, DEG_W), jnp.float32)])
    zeros128 = jnp.zeros((CHUNK, D), jnp.float32)

    degp = _sc_degree(dst3, aux16)

    sxw = pl.pallas_call(
        _tc_prepare_body,
        out_shape=jax.ShapeDtypeStruct((N, D), jnp.float32),
    )(embeddings, W_conv, degp)

    parts = _sc_aggregate(src3, dst3, sxw, zeros128)

    y = pl.pallas_call(
        _tc_finish_body,
        out_shape=jax.ShapeDtypeStruct((N, D), jnp.float32),
    )(parts, sxw, features, W_mlp, degp)
    return y


# speedup vs baseline: 12.4369x; 1.1898x over previous
import functools

import jax
import jax.numpy as jnp
from jax import lax
from jax.experimental import pallas as pl
from jax.experimental.pallas import tpu as pltpu
from jax.experimental.pallas import tpu_sc as plsc

N = 10000
E = 320000
D = 128
KAPPA = 0.95

NC = 2
NS = 16
NW = NC * NS
CHUNK = 128
EPAD = 327680
CPT = EPAD // (NW * CHUNK)
ACC_ROWS = 10240
DUMMY = N
DEG_W = 16
ROWS_PER_TILE = ACC_ROWS // NS
FIRE = 8

_mesh = plsc.VectorSubcoreMesh(core_axis_name="c", subcore_axis_name="s")


@functools.partial(
    pl.kernel,
    out_type=jax.ShapeDtypeStruct((NC, ACC_ROWS, DEG_W), jnp.float32),
    mesh=_mesh,
    scratch_types=[
        pltpu.VMEM((CHUNK, DEG_W), jnp.float32),
        pltpu.VMEM((CHUNK, DEG_W), jnp.float32),
        pltpu.VMEM((CPT, CHUNK), jnp.int32),
        pltpu.VMEM_SHARED((ACC_ROWS, DEG_W), jnp.float32),
        pltpu.SemaphoreType.DMA,
    ],
)
def _sc_degree(dst_hbm, aux_hbm, out_hbm, zero_v, ones_v, idx_v, acc, sem):
    c = lax.axis_index("c")
    s = lax.axis_index("s")
    wid = s * NC + c
    pltpu.sync_copy(aux_hbm.at[0], zero_v)
    pltpu.sync_copy(aux_hbm.at[1], ones_v)
    pltpu.sync_copy(dst_hbm.at[wid], idx_v)
    base = s * ROWS_PER_TILE
    for k in range(ROWS_PER_TILE // CHUNK):
        pltpu.sync_copy(zero_v, acc.at[pl.ds(base + k * CHUNK, CHUNK)])
    plsc.subcore_barrier()

    def body(b, carry):
        j0 = b * FIRE
        for k in range(FIRE):
            pltpu.async_copy(ones_v, acc.at[idx_v.at[j0 + k]], sem, add=True)
        for k in range(FIRE):
            pltpu.make_async_copy(ones_v, acc.at[idx_v.at[j0 + k]], sem).wait()
        return carry

    lax.fori_loop(0, CPT // FIRE, body, 0)
    plsc.subcore_barrier()
    pltpu.sync_copy(acc.at[pl.ds(base, ROWS_PER_TILE)],
                    out_hbm.at[c, pl.ds(base, ROWS_PER_TILE)])


@functools.partial(
    pl.kernel,
    out_type=jax.ShapeDtypeStruct((NC, ACC_ROWS, D), jnp.float32),
    mesh=_mesh,
    scratch_types=[
        pltpu.VMEM((2, CHUNK), jnp.int32),
        pltpu.VMEM((2, CHUNK), jnp.int32),
        pltpu.VMEM((CHUNK, D), jnp.float32),
        pltpu.VMEM((CHUNK, D), jnp.float32),
        pltpu.VMEM_SHARED((ACC_ROWS, D), jnp.float32),
        pltpu.SemaphoreType.DMA,
        pltpu.SemaphoreType.DMA,
        pltpu.SemaphoreType.DMA,
    ],
)
def _sc_aggregate(edges_hbm, sxw_hbm, zeros_hbm, out_hbm,
                  idx0, idx1, rows0, rows1, acc, sem0, sem1, semi):
    c = lax.axis_index("c")
    s = lax.axis_index("s")
    wid = s * NC + c
    pltpu.sync_copy(edges_hbm.at[wid, 0], idx0)
    pltpu.async_copy(edges_hbm.at[wid, 1], idx1, semi)
    pltpu.async_copy(sxw_hbm.at[idx0.at[0]], rows0, sem0)
    pltpu.sync_copy(zeros_hbm, rows1)
    base = s * ROWS_PER_TILE
    for k in range(ROWS_PER_TILE // CHUNK):
        pltpu.sync_copy(rows1, acc.at[pl.ds(base + k * CHUNK, CHUNK)])
    plsc.subcore_barrier()

    def body(t, carry):
        j0 = 2 * t
        not_last = t + 1 < CPT // 2
        pltpu.make_async_copy(edges_hbm.at[wid, j0 + 1], idx1, semi).wait()
        pltpu.async_copy(sxw_hbm.at[idx1.at[0]], rows1, sem1)
        pltpu.make_async_copy(sxw_hbm.at[idx0.at[0]], rows0, sem0).wait()
        pltpu.sync_copy(rows0, acc.at[idx0.at[1]], add=True)

        @pl.when(not_last)
        def _():
            pltpu.sync_copy(edges_hbm.at[wid, j0 + 2], idx0)
            pltpu.async_copy(sxw_hbm.at[idx0.at[0]], rows0, sem0)

        pltpu.make_async_copy(sxw_hbm.at[idx1.at[0]], rows1, sem1).wait()
        pltpu.sync_copy(rows1, acc.at[idx1.at[1]], add=True)

        @pl.when(not_last)
        def _():
            pltpu.async_copy(edges_hbm.at[wid, j0 + 3], idx1, semi)

        return carry

    lax.fori_loop(0, CPT // 2, body, 0)
    plsc.subcore_barrier()
    pltpu.sync_copy(acc.at[pl.ds(base, ROWS_PER_TILE)],
                    out_hbm.at[c, pl.ds(base, ROWS_PER_TILE)])


def _project(W):
    Wabs = jnp.abs(W)
    row_sum = jnp.sum(Wabs, axis=1, keepdims=True)
    hi0 = jnp.max(Wabs, axis=1, keepdims=True)

    def bis(i, lohi):
        lo, hi = lohi
        mid = 0.5 * (lo + hi)
        g = jnp.sum(jnp.maximum(Wabs - mid, 0.0), axis=1, keepdims=True)
        gt = g > KAPPA
        return (jnp.where(gt, mid, lo), jnp.where(gt, hi, mid))

    lo, hi = lax.fori_loop(0, 40, bis, (jnp.zeros_like(hi0), hi0))
    theta = 0.5 * (lo + hi)
    proj = jnp.sign(W) * jnp.maximum(Wabs - theta, 0.0)
    return jnp.where(row_sum > KAPPA, proj, W)


def _dinv_from_parts(degp_ref):
    deg = degp_ref[0, 0:N, 0:1] + degp_ref[1, 0:N, 0:1] + 1.0
    return lax.rsqrt(deg)


def _tc_prepare_body(emb_ref, w_ref, degp_ref, out_ref):
    Wc = _project(w_ref[...])
    dinv = _dinv_from_parts(degp_ref)
    xw = lax.dot_general(emb_ref[...], Wc, (((1,), (1,)), ((), ())),
                         preferred_element_type=jnp.float32)
    out_ref[...] = xw * dinv


def _tc_finish_body(parts_ref, sxw_ref, feat_ref, wmlp_ref, degp_ref, out_ref):
    dinv = _dinv_from_parts(degp_ref)
    agg = parts_ref[0, 0:N] + parts_ref[1, 0:N] + sxw_ref[...]
    mlp = lax.dot_general(feat_ref[...], wmlp_ref[...],
                          (((1,), (1,)), ((), ())),
                          preferred_element_type=jnp.float32)
    out_ref[...] = jnp.maximum(agg * dinv + mlp, 0.0)


def kernel(features, sparse_adj, embeddings, W_conv, W_mlp):
    src = sparse_adj[0]
    dst = sparse_adj[1]
    pad = EPAD - E
    src_p = jnp.concatenate([src, jnp.zeros((pad,), jnp.int32)])
    dst_p = jnp.concatenate([dst, jnp.full((pad,), DUMMY, jnp.int32)])
    src3 = src_p.reshape(NW, CPT, 1, CHUNK)
    dst3 = dst_p.reshape(NW, CPT, 1, CHUNK)
    edges = jnp.concatenate([src3, dst3], axis=2)

    aux16 = jnp.stack([jnp.zeros((CHUNK, DEG_W), jnp.float32),
                       jnp.ones((CHUNK, DEG_W), jnp.float32)])
    zeros128 = jnp.zeros((CHUNK, D), jnp.float32)

    degp = _sc_degree(dst3.reshape(NW, CPT, CHUNK), aux16)

    sxw = pl.pallas_call(
        _tc_prepare_body,
        out_shape=jax.ShapeDtypeStruct((N, D), jnp.float32),
    )(embeddings, W_conv, degp)

    parts = _sc_aggregate(edges, sxw, zeros128)

    y = pl.pallas_call(
        _tc_finish_body,
        out_shape=jax.ShapeDtypeStruct((N, D), jnp.float32),
    )(parts, sxw, features, W_mlp, degp)
    return y


# speedup vs baseline: 15.1905x; 1.2214x over previous
import functools

import jax
import jax.numpy as jnp
from jax import lax
from jax.experimental import pallas as pl
from jax.experimental.pallas import tpu as pltpu
from jax.experimental.pallas import tpu_sc as plsc

N = 10000
E = 320000
D = 128
KAPPA = 0.95

NC = 2
NS = 16
NW = NC * NS
CHUNK = 128
EPAD = 327680
CPT = EPAD // (NW * CHUNK)
CPT0 = 128
CPT1 = 32
ACC_ROWS = 10240
DUMMY = N
DEG_W = 16
ROWS_PER_TILE = ACC_ROWS // NS
FIRE = 8

_mesh = plsc.VectorSubcoreMesh(core_axis_name="c", subcore_axis_name="s")


@functools.partial(
    pl.kernel,
    out_type=jax.ShapeDtypeStruct((NC, ACC_ROWS, DEG_W), jnp.float32),
    mesh=_mesh,
    scratch_types=[
        pltpu.VMEM((CHUNK, DEG_W), jnp.float32),
        pltpu.VMEM((CHUNK, DEG_W), jnp.float32),
        pltpu.VMEM((CPT, CHUNK), jnp.int32),
        pltpu.VMEM_SHARED((ACC_ROWS, DEG_W), jnp.float32),
        pltpu.SemaphoreType.DMA,
    ],
)
def _sc_degree(dst_hbm, aux_hbm, out_hbm, zero_v, ones_v, idx_v, acc, sem):
    c = lax.axis_index("c")
    s = lax.axis_index("s")
    wid = s * NC + c
    pltpu.sync_copy(aux_hbm.at[0], zero_v)
    pltpu.sync_copy(aux_hbm.at[1], ones_v)
    pltpu.sync_copy(dst_hbm.at[wid], idx_v)
    base = s * ROWS_PER_TILE
    for k in range(ROWS_PER_TILE // CHUNK):
        pltpu.sync_copy(zero_v, acc.at[pl.ds(base + k * CHUNK, CHUNK)])
    plsc.subcore_barrier()

    def body(b, carry):
        j0 = b * FIRE
        for k in range(FIRE):
            pltpu.async_copy(ones_v, acc.at[idx_v.at[j0 + k]], sem, add=True)
        for k in range(FIRE):
            pltpu.make_async_copy(ones_v, acc.at[idx_v.at[j0 + k]], sem).wait()
        return carry

    lax.fori_loop(0, CPT // FIRE, body, 0)
    plsc.subcore_barrier()
    pltpu.sync_copy(acc.at[pl.ds(base, ROWS_PER_TILE)],
                    out_hbm.at[c, pl.ds(base, ROWS_PER_TILE)])


@functools.partial(
    pl.kernel,
    out_type=jax.ShapeDtypeStruct((NC, ACC_ROWS, D), jnp.float32),
    mesh=_mesh,
    scratch_types=[
        pltpu.VMEM((2, CHUNK), jnp.int32),
        pltpu.VMEM((2, CHUNK), jnp.int32),
        pltpu.VMEM((CHUNK, D), jnp.float32),
        pltpu.VMEM((CHUNK, D), jnp.float32),
        pltpu.VMEM_SHARED((ACC_ROWS, D), jnp.float32),
        pltpu.SemaphoreType.DMA,
        pltpu.SemaphoreType.DMA,
        pltpu.SemaphoreType.DMA,
    ],
)
def _sc_aggregate(edges_hbm, sxw_hbm, zeros_hbm, out_hbm,
                  idx0, idx1, rows0, rows1, acc, sem0, sem1, semi):
    c = lax.axis_index("c")
    s = lax.axis_index("s")
    wid = c * NS + s
    npairs = jnp.where(c == 0, CPT0 // 2, CPT1 // 2)
    pltpu.sync_copy(edges_hbm.at[wid, 0], idx0)
    pltpu.async_copy(edges_hbm.at[wid, 1], idx1, semi)
    pltpu.async_copy(sxw_hbm.at[idx0.at[0]], rows0, sem0)
    pltpu.sync_copy(zeros_hbm, rows1)
    base = s * ROWS_PER_TILE
    for k in range(ROWS_PER_TILE // CHUNK):
        pltpu.sync_copy(rows1, acc.at[pl.ds(base + k * CHUNK, CHUNK)])
    plsc.subcore_barrier()

    def body(t, carry):
        j0 = 2 * t
        not_last = t + 1 < npairs
        pltpu.make_async_copy(edges_hbm.at[wid, j0 + 1], idx1, semi).wait()
        pltpu.async_copy(sxw_hbm.at[idx1.at[0]], rows1, sem1)
        pltpu.make_async_copy(sxw_hbm.at[idx0.at[0]], rows0, sem0).wait()
        pltpu.sync_copy(rows0, acc.at[idx0.at[1]], add=True)

        @pl.when(not_last)
        def _():
            pltpu.sync_copy(edges_hbm.at[wid, j0 + 2], idx0)
            pltpu.async_copy(sxw_hbm.at[idx0.at[0]], rows0, sem0)

        pltpu.make_async_copy(sxw_hbm.at[idx1.at[0]], rows1, sem1).wait()
        pltpu.sync_copy(rows1, acc.at[idx1.at[1]], add=True)

        @pl.when(not_last)
        def _():
            pltpu.async_copy(edges_hbm.at[wid, j0 + 3], idx1, semi)

        return carry

    lax.fori_loop(0, npairs, body, 0)
    plsc.subcore_barrier()
    pltpu.sync_copy(acc.at[pl.ds(base, ROWS_PER_TILE)],
                    out_hbm.at[c, pl.ds(base, ROWS_PER_TILE)])


def _project(W):
    Wabs = jnp.abs(W)
    row_sum = jnp.sum(Wabs, axis=1, keepdims=True)
    hi0 = jnp.max(Wabs, axis=1, keepdims=True)

    def bis(i, lohi):
        lo, hi = lohi
        mid = 0.5 * (lo + hi)
        g = jnp.sum(jnp.maximum(Wabs - mid, 0.0), axis=1, keepdims=True)
        gt = g > KAPPA
        return (jnp.where(gt, mid, lo), jnp.where(gt, hi, mid))

    lo, hi = lax.fori_loop(0, 40, bis, (jnp.zeros_like(hi0), hi0))
    theta = 0.5 * (lo + hi)
    proj = jnp.sign(W) * jnp.maximum(Wabs - theta, 0.0)
    return jnp.where(row_sum > KAPPA, proj, W)


def _dinv_from_parts(degp_ref):
    deg = degp_ref[0, 0:N, 0:1] + degp_ref[1, 0:N, 0:1] + 1.0
    return lax.rsqrt(deg)


def _tc_prepare_body(emb_ref, w_ref, degp_ref, out_ref):
    Wc = _project(w_ref[...])
    dinv = _dinv_from_parts(degp_ref)
    xw = lax.dot_general(emb_ref[...], Wc, (((1,), (1,)), ((), ())),
                         preferred_element_type=jnp.float32)
    out_ref[...] = xw * dinv


def _tc_finish_body(parts_ref, sxw_ref, feat_ref, wmlp_ref, degp_ref, out_ref):
    dinv = _dinv_from_parts(degp_ref)
    agg = parts_ref[0, 0:N] + parts_ref[1, 0:N] + sxw_ref[...]
    mlp = lax.dot_general(feat_ref[...], wmlp_ref[...],
                          (((1,), (1,)), ((), ())),
                          preferred_element_type=jnp.float32)
    out_ref[...] = jnp.maximum(agg * dinv + mlp, 0.0)


def kernel(features, sparse_adj, embeddings, W_conv, W_mlp):
    src = sparse_adj[0]
    dst = sparse_adj[1]
    pad = EPAD - E
    src_p = jnp.concatenate([src, jnp.zeros((pad,), jnp.int32)])
    dst_p = jnp.concatenate([dst, jnp.full((pad,), DUMMY, jnp.int32)])
    src3 = src_p.reshape(NW, CPT, 1, CHUNK)
    dst3 = dst_p.reshape(NW, CPT, 1, CHUNK)
    n0 = NS * CPT0 * CHUNK
    e0 = src_p[:n0].reshape(NS, CPT0, 1, CHUNK)
    d0 = dst_p[:n0].reshape(NS, CPT0, 1, CHUNK)
    e1 = src_p[n0:].reshape(NS, CPT1, 1, CHUNK)
    d1 = dst_p[n0:].reshape(NS, CPT1, 1, CHUNK)
    pad_c = ((0, 0), (0, CPT0 - CPT1), (0, 0), (0, 0))
    edges = jnp.concatenate(
        [jnp.concatenate([e0, d0], axis=2),
         jnp.pad(jnp.concatenate([e1, d1], axis=2), pad_c)], axis=0)

    aux16 = jnp.stack([jnp.zeros((CHUNK, DEG_W), jnp.float32),
                       jnp.ones((CHUNK, DEG_W), jnp.float32)])
    zeros128 = jnp.zeros((CHUNK, D), jnp.float32)

    degp = _sc_degree(dst3.reshape(NW, CPT, CHUNK), aux16)

    sxw = pl.pallas_call(
        _tc_prepare_body,
        out_shape=jax.ShapeDtypeStruct((N, D), jnp.float32),
    )(embeddings, W_conv, degp)

    parts = _sc_aggregate(edges, sxw, zeros128)

    y = pl.pallas_call(
        _tc_finish_body,
        out_shape=jax.ShapeDtypeStruct((N, D), jnp.float32),
    )(parts, sxw, features, W_mlp, degp)
    return y


# speedup vs baseline: 37.4685x; 2.4666x over previous
import functools

import jax
import jax.numpy as jnp
from jax import lax
from jax.experimental import pallas as pl
from jax.experimental.pallas import tpu as pltpu
from jax.experimental.pallas import tpu_sc as plsc

N = 10000
E = 320000
D = 128
KAPPA = 0.95

NC = 2
NS = 16
NW = NC * NS
CHUNK = 128
EPAD = 327680
CPT = EPAD // (NW * CHUNK)
ACC_ROWS = 10240
DUMMY = N
DEG_W = 16
ROWS_PER_TILE = ACC_ROWS // NS
FIRE = 8

_mesh = plsc.VectorSubcoreMesh(core_axis_name="c", subcore_axis_name="s")


@functools.partial(
    pl.kernel,
    out_type=jax.ShapeDtypeStruct((NC, ACC_ROWS, DEG_W), jnp.float32),
    mesh=_mesh,
    scratch_types=[
        pltpu.VMEM((CHUNK, DEG_W), jnp.float32),
        pltpu.VMEM((CHUNK, DEG_W), jnp.float32),
        pltpu.VMEM((CPT, CHUNK), jnp.int32),
        pltpu.VMEM_SHARED((ACC_ROWS, DEG_W), jnp.float32),
        pltpu.SemaphoreType.DMA,
    ],
)
def _sc_degree(dst_hbm, aux_hbm, out_hbm, zero_v, ones_v, idx_v, acc, sem):
    c = lax.axis_index("c")
    s = lax.axis_index("s")
    wid = s * NC + c
    pltpu.sync_copy(aux_hbm.at[0], zero_v)
    pltpu.sync_copy(aux_hbm.at[1], ones_v)
    pltpu.sync_copy(dst_hbm.at[wid], idx_v)
    base = s * ROWS_PER_TILE
    for k in range(ROWS_PER_TILE // CHUNK):
        pltpu.sync_copy(zero_v, acc.at[pl.ds(base + k * CHUNK, CHUNK)])
    plsc.subcore_barrier()

    def body(b, carry):
        j0 = b * FIRE
        for k in range(FIRE):
            pltpu.async_copy(ones_v, acc.at[idx_v.at[j0 + k]], sem, add=True)
        for k in range(FIRE):
            pltpu.make_async_copy(ones_v, acc.at[idx_v.at[j0 + k]], sem).wait()
        return carry

    lax.fori_loop(0, CPT // FIRE, body, 0)
    plsc.subcore_barrier()
    pltpu.sync_copy(acc.at[pl.ds(base, ROWS_PER_TILE)],
                    out_hbm.at[c, pl.ds(base, ROWS_PER_TILE)])


@functools.partial(
    pl.kernel,
    out_type=jax.ShapeDtypeStruct((NC, ACC_ROWS, D), jnp.float32),
    mesh=_mesh,
    scratch_types=[
        pltpu.VMEM((2, CHUNK), jnp.int32),
        pltpu.VMEM((2, CHUNK), jnp.int32),
        pltpu.VMEM((CHUNK, D), jnp.float32),
        pltpu.VMEM((CHUNK, D), jnp.float32),
        pltpu.VMEM_SHARED((ACC_ROWS, D), jnp.float32),
        pltpu.SemaphoreType.DMA,
        pltpu.SemaphoreType.DMA,
        pltpu.SemaphoreType.DMA,
    ],
)
def _sc_aggregate(edges_hbm, sxw_hbm, zeros_hbm, out_hbm,
                  idx0, idx1, rows0, rows1, acc, sem0, sem1, semi):
    c = lax.axis_index("c")
    s = lax.axis_index("s")
    wid = s * NC + c
    npairs = CPT // 2
    pltpu.sync_copy(edges_hbm.at[wid, 0], idx0)
    pltpu.async_copy(edges_hbm.at[wid, 1], idx1, semi)
    pltpu.async_copy(sxw_hbm.at[idx0.at[0]], rows0, sem0)
    pltpu.sync_copy(zeros_hbm, rows1)
    base = s * ROWS_PER_TILE
    for k in range(ROWS_PER_TILE // CHUNK):
        pltpu.sync_copy(rows1, acc.at[pl.ds(base + k * CHUNK, CHUNK)])
    plsc.subcore_barrier()

    def body(t, carry):
        j0 = 2 * t
        not_last = t + 1 < npairs
        pltpu.make_async_copy(edges_hbm.at[wid, j0 + 1], idx1, semi).wait()
        pltpu.async_copy(sxw_hbm.at[idx1.at[0]], rows1, sem1)
        pltpu.make_async_copy(sxw_hbm.at[idx0.at[0]], rows0, sem0).wait()
        pltpu.sync_copy(rows0, acc.at[idx0.at[1]], add=True)

        @pl.when(not_last)
        def _():
            pltpu.sync_copy(edges_hbm.at[wid, j0 + 2], idx0)
            pltpu.async_copy(sxw_hbm.at[idx0.at[0]], rows0, sem0)

        pltpu.make_async_copy(sxw_hbm.at[idx1.at[0]], rows1, sem1).wait()
        pltpu.sync_copy(rows1, acc.at[idx1.at[1]], add=True)

        @pl.when(not_last)
        def _():
            pltpu.async_copy(edges_hbm.at[wid, j0 + 3], idx1, semi)

        return carry

    lax.fori_loop(0, npairs, body, 0)
    plsc.subcore_barrier()
    pltpu.sync_copy(acc.at[pl.ds(base, ROWS_PER_TILE)],
                    out_hbm.at[c, pl.ds(base, ROWS_PER_TILE)])


def _project(W):
    Wabs = jnp.abs(W)
    row_sum = jnp.sum(Wabs, axis=1, keepdims=True)
    hi0 = jnp.max(Wabs, axis=1, keepdims=True)

    def bis(i, lohi):
        lo, hi = lohi
        mid = 0.5 * (lo + hi)
        g = jnp.sum(jnp.maximum(Wabs - mid, 0.0), axis=1, keepdims=True)
        gt = g > KAPPA
        return (jnp.where(gt, mid, lo), jnp.where(gt, hi, mid))

    lo, hi = lax.fori_loop(0, 40, bis, (jnp.zeros_like(hi0), hi0))
    theta = 0.5 * (lo + hi)
    proj = jnp.sign(W) * jnp.maximum(Wabs - theta, 0.0)
    return jnp.where(row_sum > KAPPA, proj, W)


def _dinv_from_parts(degp_ref):
    deg = degp_ref[0, 0:N, 0:1] + degp_ref[1, 0:N, 0:1] + 1.0
    return lax.rsqrt(deg)


def _tc_prepare_body(emb_ref, w_ref, degp_ref, out_ref):
    Wc = _project(w_ref[...])
    dinv = _dinv_from_parts(degp_ref)
    xw = lax.dot_general(emb_ref[...], Wc, (((1,), (1,)), ((), ())),
                         preferred_element_type=jnp.float32)
    out_ref[...] = xw * dinv


def _tc_finish_body(parts_ref, sxw_ref, feat_ref, wmlp_ref, degp_ref, out_ref):
    dinv = _dinv_from_parts(degp_ref)
    agg = parts_ref[0, 0:N] + parts_ref[1, 0:N] + sxw_ref[...]
    mlp = lax.dot_general(feat_ref[...], wmlp_ref[...],
                          (((1,), (1,)), ((), ())),
                          preferred_element_type=jnp.float32)
    out_ref[...] = jnp.maximum(agg * dinv + mlp, 0.0)


def kernel(features, sparse_adj, embeddings, W_conv, W_mlp):
    src = sparse_adj[0]
    dst = sparse_adj[1]
    pad = EPAD - E
    pad_idx = jnp.arange(pad, dtype=jnp.int32)
    src_p = jnp.concatenate([src, pad_idx % N])
    dst_p = jnp.concatenate([dst, N + pad_idx % (ACC_ROWS - N)])
    src3 = src_p.reshape(NW, CPT, 1, CHUNK)
    dst3 = dst_p.reshape(NW, CPT, 1, CHUNK)
    edges = jnp.concatenate([src3, dst3], axis=2)

    aux16 = jnp.stack([jnp.zeros((CHUNK, DEG_W), jnp.float32),
                       jnp.ones((CHUNK, DEG_W), jnp.float32)])
    zeros128 = jnp.zeros((CHUNK, D), jnp.float32)

    degp = _sc_degree(dst3.reshape(NW, CPT, CHUNK), aux16)

    sxw = pl.pallas_call(
        _tc_prepare_body,
        out_shape=jax.ShapeDtypeStruct((N, D), jnp.float32),
    )(embeddings, W_conv, degp)

    parts = _sc_aggregate(edges, sxw, zeros128)

    y = pl.pallas_call(
        _tc_finish_body,
        out_shape=jax.ShapeDtypeStruct((N, D), jnp.float32),
    )(parts, sxw, features, W_mlp, degp)
    return y
